# baseline (speedup 1.0000x reference)
import jax
import jax.numpy as jnp
from jax.experimental import pallas as pl
from jax.experimental.pallas import tpu as pltpu


def _graph_conv_body(x_ref, w1_ref, adj_ref, b_ref, o_ref, w_scr):
    @pl.when(pl.program_id(0) == 0)
    def _():
        w_scr[...] = jnp.dot(
            w1_ref[...], adj_ref[...], preferred_element_type=jnp.float32
        )

    o_ref[...] = (
        jnp.dot(x_ref[...], w_scr[...], preferred_element_type=jnp.float32)
        + b_ref[...]
    )


def kernel(x, adj, W1, b):
    n, din = x.shape
    dout = adj.shape[1]
    blk = 2000
    assert n % blk == 0

    return pl.pallas_call(
        _graph_conv_body,
        grid=(n // blk,),
        in_specs=[
            pl.BlockSpec((blk, din), lambda i: (i, 0)),
            pl.BlockSpec((din, dout), lambda i: (0, 0)),
            pl.BlockSpec((dout, dout), lambda i: (0, 0)),
            pl.BlockSpec((1, dout), lambda i: (0, 0)),
        ],
        out_specs=pl.BlockSpec((blk, dout), lambda i: (i, 0)),
        out_shape=jax.ShapeDtypeStruct((n, dout), x.dtype),
        scratch_shapes=[pltpu.VMEM((din, dout), jnp.float32)],
    )(x, W1, adj, b.reshape(1, dout))
